# Initial kernel scaffold; baseline (speedup 1.0000x reference)
import jax
import jax.numpy as jnp
from jax.experimental import pallas as pl


def kernel(n, relative_attention_bias):
    raise NotImplementedError("write your pallas kernel here")


# speedup vs baseline: 127.5072x; 127.5072x over previous
import math

import jax
import jax.numpy as jnp
from jax.experimental import pallas as pl
from jax.experimental.pallas import tpu as pltpu

_HEADS = 16
_N = 2048
_BLK = 256
_W8 = 4352
_W256 = 4096


def _fill_kernel(table_ref, out_ref, v8_ref, v256_ref):
    h = pl.program_id(0)
    bi = pl.program_id(1)

    @pl.when(jnp.logical_and(h == 0, bi == 0))
    def _build_v8():
        s_io = jax.lax.broadcasted_iota(jnp.int32, (8, _W8), 0)
        d_io = jax.lax.broadcasted_iota(jnp.int32, (8, _W8), 1)
        rel = d_io - s_io - 2175
        n = -rel
        ret = jnp.where(n < 0, 16, 0)
        na = jnp.abs(n)
        is_small = na < 8
        naf = jnp.maximum(na, 8).astype(jnp.float32)
        val_large = 8 + (
            jnp.log(naf / 8.0) / math.log(128.0 / 8.0) * 8.0
        ).astype(jnp.int32)
        val_large = jnp.minimum(val_large, 15)
        bucket = ret + jnp.where(is_small, na, val_large)
        for hh in range(_HEADS):
            acc = jax.lax.fori_loop(
                0,
                32,
                lambda b, a: jnp.where(bucket == b, table_ref[b, hh], a),
                jnp.zeros((8, _W8), jnp.float32),
            )
            v8_ref[hh] = acc

    @pl.when(bi == 0)
    def _build_v256():
        for k in range(_BLK // 8):
            off = 255 - 8 * k
            v256_ref[8 * k : 8 * (k + 1), :] = v8_ref[h, :, off : off + _W256]

    start = pl.multiple_of(1920 - _BLK * bi, 128)
    out_ref[0, :, :] = v256_ref[:, pl.ds(start, _N)]


def kernel(n, relative_attention_bias):
    del n
    return pl.pallas_call(
        _fill_kernel,
        grid=(_HEADS, _N // _BLK),
        in_specs=[
            pl.BlockSpec((32, _HEADS), lambda h, b: (0, 0),
                         memory_space=pltpu.SMEM),
        ],
        out_specs=pl.BlockSpec((1, _BLK, _N), lambda h, b: (h, b, 0)),
        out_shape=jax.ShapeDtypeStruct((_HEADS, _N, _N), jnp.float32),
        scratch_shapes=[
            pltpu.VMEM((_HEADS, 8, _W8), jnp.float32),
            pltpu.VMEM((_BLK, _W256), jnp.float32),
        ],
    )(relative_attention_bias)


# speedup vs baseline: 167.2024x; 1.3113x over previous
import math

import jax
import jax.numpy as jnp
from jax.experimental import pallas as pl
from jax.experimental.pallas import tpu as pltpu

_HEADS = 16
_N = 2048
_BLK = 256
_W8 = 4352
_W256 = 4096


def _fill_kernel(table_ref, out_ref, v8_ref, v256_ref, sem_ref):
    h = pl.program_id(0)
    p = jax.lax.rem(h, 2)

    @pl.when(h == 0)
    def _build_v8():
        s_io = jax.lax.broadcasted_iota(jnp.int32, (8, _W8), 0)
        d_io = jax.lax.broadcasted_iota(jnp.int32, (8, _W8), 1)
        rel = d_io - s_io - 2175
        n = -rel
        ret = jnp.where(n < 0, 16, 0)
        na = jnp.abs(n)
        is_small = na < 8
        naf = jnp.maximum(na, 8).astype(jnp.float32)
        val_large = 8 + (
            jnp.log(naf / 8.0) / math.log(128.0 / 8.0) * 8.0
        ).astype(jnp.int32)
        val_large = jnp.minimum(val_large, 15)
        bucket = ret + jnp.where(is_small, na, val_large)
        for hh in range(_HEADS):
            acc = jax.lax.fori_loop(
                0,
                32,
                lambda b, a: jnp.where(bucket == b, table_ref[b, hh], a),
                jnp.zeros((8, _W8), jnp.float32),
            )
            v8_ref[hh] = acc

    def _copy(src_parity, head, g):
        return pltpu.make_async_copy(
            v256_ref.at[src_parity, :, 1920 - _BLK * g : 1920 - _BLK * g + _N],
            out_ref.at[head, pl.ds(_BLK * g, _BLK), :],
            sem_ref.at[src_parity],
        )

    @pl.when(h >= 2)
    def _drain_prev():
        for g in range(_N // _BLK):
            _copy(p, h - 2, g).wait()

    for k in range(_BLK // 8):
        off = 255 - 8 * k
        v256_ref[p, 8 * k : 8 * (k + 1), :] = v8_ref[h, :, off : off + _W256]

    for g in range(_N // _BLK):
        _copy(p, h, g).start()

    @pl.when(h == _HEADS - 1)
    def _drain_tail():
        for g in range(_N // _BLK):
            _copy(1 - p, h - 1, g).wait()
        for g in range(_N // _BLK):
            _copy(p, h, g).wait()


def kernel(n, relative_attention_bias):
    del n
    return pl.pallas_call(
        _fill_kernel,
        grid=(_HEADS,),
        in_specs=[
            pl.BlockSpec((32, _HEADS), lambda h: (0, 0),
                         memory_space=pltpu.SMEM),
        ],
        out_specs=pl.BlockSpec(memory_space=pl.ANY),
        out_shape=jax.ShapeDtypeStruct((_HEADS, _N, _N), jnp.float32),
        scratch_shapes=[
            pltpu.VMEM((_HEADS, 8, _W8), jnp.float32),
            pltpu.VMEM((2, _BLK, _W256), jnp.float32),
            pltpu.SemaphoreType.DMA((2,)),
        ],
    )(relative_attention_bias)


# speedup vs baseline: 189.9391x; 1.1360x over previous
import math

import jax
import jax.numpy as jnp
from jax.experimental import pallas as pl
from jax.experimental.pallas import tpu as pltpu

_HEADS = 16
_N = 2048
_QROWS = 128
_NQ = _N // _QROWS
_W8 = 4352
_WA = 4096


def _fill_kernel(table_ref, out_ref, bkt_ref, v8_ref, va_ref, sem_ref):
    h = pl.program_id(0)
    p = jax.lax.rem(h, 2)

    @pl.when(h == 0)
    def _build_bucket():
        s_io = jax.lax.broadcasted_iota(jnp.int32, (8, _W8), 0)
        d_io = jax.lax.broadcasted_iota(jnp.int32, (8, _W8), 1)
        rel = d_io - s_io - 2175
        n = -rel
        ret = jnp.where(n < 0, 16, 0)
        na = jnp.abs(n)
        is_small = na < 8
        naf = jnp.maximum(na, 8).astype(jnp.float32)
        val_large = 8 + (
            jnp.log(naf / 8.0) / math.log(128.0 / 8.0) * 8.0
        ).astype(jnp.int32)
        val_large = jnp.minimum(val_large, 15)
        bkt_ref[...] = ret + jnp.where(is_small, na, val_large)

    def _copy(src_parity, head, q):
        start = 1920 - _QROWS * q
        return pltpu.make_async_copy(
            va_ref.at[src_parity, :, start : start + _N],
            out_ref.at[head, pl.ds(_QROWS * q, _QROWS), :],
            sem_ref.at[src_parity],
        )

    @pl.when(h >= 2)
    def _drain_prev():
        for q in range(_NQ):
            _copy(p, h - 2, q).wait()

    bucket = bkt_ref[...]
    v8_ref[...] = jax.lax.fori_loop(
        0,
        32,
        lambda b, a: jnp.where(bucket == b, table_ref[b, h], a),
        jnp.zeros((8, _W8), jnp.float32),
    )

    for k in range(_QROWS // 8):
        off = 255 - 8 * k
        va_ref[p, 8 * k : 8 * (k + 1), :] = v8_ref[:, off : off + _WA]

    for q in range(_NQ):
        _copy(p, h, q).start()

    @pl.when(h == _HEADS - 1)
    def _drain_tail():
        for q in range(_NQ):
            _copy(1 - p, h - 1, q).wait()
        for q in range(_NQ):
            _copy(p, h, q).wait()


def kernel(n, relative_attention_bias):
    del n
    return pl.pallas_call(
        _fill_kernel,
        grid=(_HEADS,),
        in_specs=[
            pl.BlockSpec((32, _HEADS), lambda h: (0, 0),
                         memory_space=pltpu.SMEM),
        ],
        out_specs=pl.BlockSpec(memory_space=pl.ANY),
        out_shape=jax.ShapeDtypeStruct((_HEADS, _N, _N), jnp.float32),
        scratch_shapes=[
            pltpu.VMEM((8, _W8), jnp.int32),
            pltpu.VMEM((8, _W8), jnp.float32),
            pltpu.VMEM((2, _QROWS, _WA), jnp.float32),
            pltpu.SemaphoreType.DMA((2,)),
        ],
    )(relative_attention_bias)
